# baseline (speedup 1.0000x reference)
import functools

import jax
import jax.numpy as jnp
from jax import lax
from jax.experimental import pallas as pl
from jax.experimental.pallas import tpu as pltpu
from jax.experimental.pallas import tpu_sc as plsc

NBUF = 2
IDXB = 16


def _make_lookup(B: int, L: int, D: int):
    info = plsc.get_sparse_core_info()
    NC, NS = info.num_cores, info.num_subcores
    NW = NC * NS
    DT, DS, BT = D // 8, 8, 128
    NBT = B // BT
    n_blocks = L * NBT
    assert n_blocks % (NW * IDXB) == 0
    blk_per_w = n_blocks // NW
    n_groups = blk_per_w // IDXB
    GSZ = IDXB * BT

    mesh = plsc.VectorSubcoreMesh(core_axis_name="c", subcore_axis_name="s")

    @functools.partial(
        pl.kernel,
        mesh=mesh,
        out_type=jax.ShapeDtypeStruct((L, DT, NBT, DS, BT), jnp.float32),
        scratch_types=[
            pltpu.VMEM((2 * GSZ,), jnp.int32),
            [pltpu.VMEM((BT, D), jnp.float32) for _ in range(NBUF)],
            [pltpu.VMEM((DT, DS, BT), jnp.float32) for _ in range(NBUF)],
            [pltpu.SemaphoreType.DMA for _ in range(NBUF)],
            [pltpu.SemaphoreType.DMA for _ in range(NBUF)],
        ],
        compiler_params=pltpu.CompilerParams(use_tc_tiling_on_sc=False,
                                             needs_layout_passes=False),
    )
    def lookup(idx_hbm, table_hbm, out_hbm, idxv, buf, tbuf, gsem, ssem):
        wid = lax.axis_index("s") * NC + lax.axis_index("c")
        blk0 = wid * blk_per_w

        iota16 = lax.iota(jnp.int32, 16)
        bvecs = [iota16 + 16 * j for j in range(8)]

        def stage_idx(g):
            off = (blk0 + g * IDXB) * BT
            pltpu.sync_copy(idx_hbm.at[pl.ds(off, GSZ)],
                            idxv.at[pl.ds((g & 1) * GSZ, GSZ)])

        def gather(blk, slot):
            g = blk >> 4
            t = blk & (IDXB - 1)
            off = (g & 1) * GSZ + t * BT
            src = table_hbm.at[idxv.at[pl.ds(off, BT)]]
            return pltpu.make_async_copy(src, buf[slot], gsem[slot])

        def store(l, bt, dt, slot):
            return pltpu.make_async_copy(tbuf[slot].at[dt],
                                         out_hbm.at[l, dt, bt], ssem[slot])

        def transpose(slot):
            for d in range(D):
                dvec = jnp.full((16,), d, jnp.int32)
                for j in range(8):
                    v = plsc.load_gather(buf[slot], [bvecs[j], dvec])
                    tbuf[slot][d // DS, d % DS, pl.ds(j * 16, 16)] = v

        def block_body(s, p):
            blk = s * 2 + p
            f = blk0 + blk
            l = f >> 7
            bt = f & (NBT - 1)
            gather(blk, p).wait()

            @pl.when(s > 0)
            def _():
                for dt in range(DT):
                    store(l, bt, dt, p).wait()

            transpose(p)
            for dt in range(DT):
                store(l, bt, dt, p).start()

            nblk = blk + NBUF

            @pl.when(nblk < blk_per_w)
            def _():
                gather(nblk, p).start()

        def pairstep(s, _):
            g = s >> 3

            @pl.when((s & 7) == 0)
            def _():
                @pl.when(g < n_groups - 1)
                def _():
                    stage_idx(g + 1)

            block_body(s, 0)
            block_body(s, 1)
            return ()

        stage_idx(jnp.int32(0))
        for p in range(NBUF):
            gather(jnp.int32(p), p).start()

        lax.fori_loop(0, blk_per_w // 2, pairstep, (), unroll=False)

        for p in range(NBUF):
            f = blk0 + blk_per_w - NBUF + p
            for dt in range(DT):
                store(f >> 7, f & (NBT - 1), dt, p).wait()

    return lookup


def kernel(p_sequences, table):
    B, L = p_sequences.shape
    V, D = table.shape
    idx_t = p_sequences.T.reshape(B * L)
    lookup = _make_lookup(B, L, D)
    x = lookup(idx_t, table)
    return x.transpose(2, 4, 0, 1, 3).reshape(B, L, D)


# speedup vs baseline: 2.5698x; 2.5698x over previous
import functools

import jax
import jax.numpy as jnp
from jax import lax
from jax.experimental import pallas as pl
from jax.experimental.pallas import tpu as pltpu
from jax.experimental.pallas import tpu_sc as plsc

NBUF = 2
IDXB = 16


def _make_lookup(B: int, L: int, D: int):
    info = plsc.get_sparse_core_info()
    NC, NS = info.num_cores, info.num_subcores
    NW = NC * NS
    DT, DS, BT = D // 8, 8, 128
    NBT = B // BT
    n_blocks = L * NBT
    assert n_blocks % (NW * IDXB) == 0
    blk_per_w = n_blocks // NW
    n_groups = blk_per_w // IDXB
    GSZ = IDXB * BT

    mesh = plsc.VectorSubcoreMesh(core_axis_name="c", subcore_axis_name="s")

    @functools.partial(
        pl.kernel,
        mesh=mesh,
        out_type=jax.ShapeDtypeStruct((L, DT, NBT, DS, BT), jnp.float32),
        scratch_types=[
            pltpu.VMEM((2 * GSZ,), jnp.int32),
            [pltpu.VMEM((BT, D), jnp.float32) for _ in range(NBUF)],
            [pltpu.VMEM((DT, DS, BT + 1), jnp.float32) for _ in range(NBUF)],
            [pltpu.SemaphoreType.DMA for _ in range(NBUF)],
            [pltpu.SemaphoreType.DMA for _ in range(NBUF)],
        ],
        compiler_params=pltpu.CompilerParams(use_tc_tiling_on_sc=False,
                                             needs_layout_passes=False),
    )
    def lookup(idx_hbm, table_hbm, out_hbm, idxv, buf, tbuf, gsem, ssem):
        wid = lax.axis_index("s") * NC + lax.axis_index("c")
        blk0 = wid * blk_per_w

        iota16 = lax.iota(jnp.int32, 16)
        bvecs = [iota16 + 16 * j for j in range(8)]

        def stage_idx(g):
            off = (blk0 + g * IDXB) * BT
            pltpu.sync_copy(idx_hbm.at[pl.ds(off, GSZ)],
                            idxv.at[pl.ds((g & 1) * GSZ, GSZ)])

        def gather(blk, slot):
            g = blk >> 4
            t = blk & (IDXB - 1)
            off = (g & 1) * GSZ + t * BT
            src = table_hbm.at[idxv.at[pl.ds(off, BT)]]
            return pltpu.make_async_copy(src, buf[slot], gsem[slot])

        def store(l, bt, dt, slot):
            return pltpu.make_async_copy(tbuf[slot].at[dt, :, pl.ds(0, BT)],
                                         out_hbm.at[l, dt, bt], ssem[slot])

        def transpose(slot):
            for q in range(D // 16):
                dq = iota16 + 16 * q
                dtq = dq >> 3
                dsq = dq & 7
                for b in range(BT):
                    v = buf[slot][b, pl.ds(16 * q, 16)]
                    plsc.store_scatter(
                        tbuf[slot], [dtq, dsq, jnp.full((16,), b, jnp.int32)],
                        v)

        def block_body(s, p):
            blk = s * 2 + p
            f = blk0 + blk
            l = f >> 7
            bt = f & (NBT - 1)
            gather(blk, p).wait()

            @pl.when(s > 0)
            def _():
                for dt in range(DT):
                    store(l, bt, dt, p).wait()

            transpose(p)
            for dt in range(DT):
                store(l, bt, dt, p).start()

            nblk = blk + NBUF

            @pl.when(nblk < blk_per_w)
            def _():
                gather(nblk, p).start()

        def pairstep(s, _):
            g = s >> 3

            @pl.when((s & 7) == 0)
            def _():
                @pl.when(g < n_groups - 1)
                def _():
                    stage_idx(g + 1)

            block_body(s, 0)
            block_body(s, 1)
            return ()

        stage_idx(jnp.int32(0))
        for p in range(NBUF):
            gather(jnp.int32(p), p).start()

        lax.fori_loop(0, blk_per_w // 2, pairstep, (), unroll=False)

        for p in range(NBUF):
            f = blk0 + blk_per_w - NBUF + p
            for dt in range(DT):
                store(f >> 7, f & (NBT - 1), dt, p).wait()

    return lookup


def kernel(p_sequences, table):
    B, L = p_sequences.shape
    V, D = table.shape
    idx_t = p_sequences.T.reshape(B * L)
    lookup = _make_lookup(B, L, D)
    x = lookup(idx_t, table)
    return x.transpose(2, 4, 0, 1, 3).reshape(B, L, D)


# speedup vs baseline: 7.2306x; 2.8137x over previous
import functools

import jax
import jax.numpy as jnp
from jax import lax
from jax.experimental import pallas as pl
from jax.experimental.pallas import tpu as pltpu
from jax.experimental.pallas import tpu_sc as plsc

NBUF = 2
IDXB = 16


def _make_lookup(B: int, L: int, D: int):
    info = plsc.get_sparse_core_info()
    NC, NS = info.num_cores, info.num_subcores
    NW = NC * NS
    DT, DS, BT = D // 8, 8, 128
    NBT = B // BT
    n_blocks = L * NBT
    assert n_blocks % (NW * IDXB) == 0
    blk_per_w = n_blocks // NW
    n_groups = blk_per_w // IDXB
    GSZ = IDXB * BT

    mesh = plsc.VectorSubcoreMesh(core_axis_name="c", subcore_axis_name="s")

    @functools.partial(
        pl.kernel,
        mesh=mesh,
        out_type=jax.ShapeDtypeStruct((L, DT, NBT, DS, BT), jnp.float32),
        scratch_types=[
            pltpu.VMEM((2 * GSZ,), jnp.int32),
            [pltpu.VMEM((BT, D), jnp.float32) for _ in range(NBUF)],
            [pltpu.VMEM((DT, DS, BT + 1), jnp.float32) for _ in range(NBUF)],
            [pltpu.SemaphoreType.DMA for _ in range(NBUF)],
            [pltpu.SemaphoreType.DMA for _ in range(NBUF)],
        ],
        compiler_params=pltpu.CompilerParams(use_tc_tiling_on_sc=False,
                                             needs_layout_passes=False),
    )
    def lookup(idx_hbm, table_hbm, out_hbm, idxv, buf, tbuf, gsem, ssem):
        wid = lax.axis_index("s") * NC + lax.axis_index("c")
        blk0 = wid * blk_per_w

        iota16 = lax.iota(jnp.int32, 16)
        dqs = [iota16 + 16 * q for q in range(D // 16)]
        dtqs = [dq >> 3 for dq in dqs]
        dsqs = [dq & 7 for dq in dqs]

        def stage_idx(g):
            off = (blk0 + g * IDXB) * BT
            pltpu.sync_copy(idx_hbm.at[pl.ds(off, GSZ)],
                            idxv.at[pl.ds((g & 1) * GSZ, GSZ)])

        def gather(blk, slot):
            g = blk >> 4
            t = blk & (IDXB - 1)
            off = (g & 1) * GSZ + t * BT
            src = table_hbm.at[idxv.at[pl.ds(off, BT)]]
            return pltpu.make_async_copy(src, buf[slot], gsem[slot])

        def store(l, bt, dt, slot):
            return pltpu.make_async_copy(tbuf[slot].at[dt, :, pl.ds(0, BT)],
                                         out_hbm.at[l, dt, bt], ssem[slot])

        def transpose(slot):
            @plsc.parallel_loop(0, BT, unroll=4)
            def _(b):
                bvec = jnp.full((16,), b, jnp.int32)
                for q in range(D // 16):
                    v = buf[slot][b, pl.ds(16 * q, 16)]
                    plsc.store_scatter(tbuf[slot], [dtqs[q], dsqs[q], bvec],
                                       v)

        def block_body(s, p):
            blk = s * 2 + p
            f = blk0 + blk
            l = f >> 7
            bt = f & (NBT - 1)
            gather(blk, p).wait()

            @pl.when(s > 0)
            def _():
                for dt in range(DT):
                    store(l, bt, dt, p).wait()

            transpose(p)
            for dt in range(DT):
                store(l, bt, dt, p).start()

            nblk = blk + NBUF

            @pl.when(nblk < blk_per_w)
            def _():
                gather(nblk, p).start()

        def pairstep(s, _):
            g = s >> 3

            @pl.when((s & 7) == 0)
            def _():
                @pl.when(g < n_groups - 1)
                def _():
                    stage_idx(g + 1)

            block_body(s, 0)
            block_body(s, 1)
            return ()

        stage_idx(jnp.int32(0))
        for p in range(NBUF):
            gather(jnp.int32(p), p).start()

        lax.fori_loop(0, blk_per_w // 2, pairstep, (), unroll=False)

        for p in range(NBUF):
            f = blk0 + blk_per_w - NBUF + p
            for dt in range(DT):
                store(f >> 7, f & (NBT - 1), dt, p).wait()

    return lookup


def kernel(p_sequences, table):
    B, L = p_sequences.shape
    V, D = table.shape
    idx_t = p_sequences.T.reshape(B * L)
    lookup = _make_lookup(B, L, D)
    x = lookup(idx_t, table)
    return x.transpose(2, 4, 0, 1, 3).reshape(B, L, D)


# speedup vs baseline: 7.2410x; 1.0014x over previous
import functools

import jax
import jax.numpy as jnp
from jax import lax
from jax.experimental import pallas as pl
from jax.experimental.pallas import tpu as pltpu
from jax.experimental.pallas import tpu_sc as plsc

NBUF = 2
IDXB = 16


def _make_lookup(B: int, L: int, D: int):
    info = plsc.get_sparse_core_info()
    NC, NS = info.num_cores, info.num_subcores
    NW = NC * NS
    DT, DS, BT = D // 8, 8, 128
    NBT = B // BT
    n_blocks = L * NBT
    assert n_blocks % (NW * IDXB) == 0
    blk_per_w = n_blocks // NW
    n_groups = blk_per_w // IDXB
    GSZ = IDXB * BT

    mesh = plsc.VectorSubcoreMesh(core_axis_name="c", subcore_axis_name="s")

    @functools.partial(
        pl.kernel,
        mesh=mesh,
        out_type=jax.ShapeDtypeStruct((L, DT, NBT, DS, BT), jnp.float32),
        scratch_types=[
            pltpu.VMEM((2 * GSZ,), jnp.int32),
            [pltpu.VMEM((BT, D), jnp.float32) for _ in range(NBUF)],
            [pltpu.VMEM((DT, DS, BT + 1), jnp.float32) for _ in range(NBUF)],
            [pltpu.SemaphoreType.DMA for _ in range(NBUF)],
            [pltpu.SemaphoreType.DMA for _ in range(NBUF)],
        ],
        compiler_params=pltpu.CompilerParams(use_tc_tiling_on_sc=False,
                                             needs_layout_passes=False),
    )
    def lookup(idx_hbm, table_hbm, out_hbm, idxv, buf, tbuf, gsem, ssem):
        wid = lax.axis_index("s") * NC + lax.axis_index("c")
        blk0 = wid * blk_per_w

        iota16 = lax.iota(jnp.int32, 16)
        dqs = [iota16 + 16 * q for q in range(D // 16)]
        dtqs = [dq >> 3 for dq in dqs]
        dsqs = [dq & 7 for dq in dqs]

        def stage_idx(g):
            off = (blk0 + g * IDXB) * BT
            pltpu.sync_copy(idx_hbm.at[pl.ds(off, GSZ)],
                            idxv.at[pl.ds((g & 1) * GSZ, GSZ)])

        def gather(blk, slot):
            g = blk >> 4
            t = blk & (IDXB - 1)
            off = (g & 1) * GSZ + t * BT
            src = table_hbm.at[idxv.at[pl.ds(off, BT)]]
            return pltpu.make_async_copy(src, buf[slot], gsem[slot])

        def store(l, bt, dt, slot):
            return pltpu.make_async_copy(tbuf[slot].at[dt, :, pl.ds(0, BT)],
                                         out_hbm.at[l, dt, bt], ssem[slot])

        def transpose(slot):
            @plsc.parallel_loop(0, BT, unroll=8)
            def _(b):
                bvec = jnp.full((16,), b, jnp.int32)
                for q in range(D // 16):
                    v = buf[slot][b, pl.ds(16 * q, 16)]
                    plsc.store_scatter(tbuf[slot], [dtqs[q], dsqs[q], bvec],
                                       v)

        def block_body(s, p):
            blk = s * 2 + p
            f = blk0 + blk
            l = f >> 7
            bt = f & (NBT - 1)
            gather(blk, p).wait()

            @pl.when(s > 0)
            def _():
                for dt in range(DT):
                    store(l, bt, dt, p).wait()

            transpose(p)
            for dt in range(DT):
                store(l, bt, dt, p).start()

            nblk = blk + NBUF

            @pl.when(nblk < blk_per_w)
            def _():
                gather(nblk, p).start()

        def pairstep(s, _):
            g = s >> 3

            @pl.when((s & 7) == 0)
            def _():
                @pl.when(g < n_groups - 1)
                def _():
                    stage_idx(g + 1)

            block_body(s, 0)
            block_body(s, 1)
            return ()

        stage_idx(jnp.int32(0))
        for p in range(NBUF):
            gather(jnp.int32(p), p).start()

        lax.fori_loop(0, blk_per_w // 2, pairstep, (), unroll=False)

        for p in range(NBUF):
            f = blk0 + blk_per_w - NBUF + p
            for dt in range(DT):
                store(f >> 7, f & (NBT - 1), dt, p).wait()

    return lookup


def kernel(p_sequences, table):
    B, L = p_sequences.shape
    V, D = table.shape
    idx_t = p_sequences.T.reshape(B * L)
    lookup = _make_lookup(B, L, D)
    x = lookup(idx_t, table)
    return x.transpose(2, 4, 0, 1, 3).reshape(B, L, D)


# speedup vs baseline: 8.1309x; 1.1229x over previous
import functools

import jax
import jax.numpy as jnp
from jax import lax
from jax.experimental import pallas as pl
from jax.experimental.pallas import tpu as pltpu
from jax.experimental.pallas import tpu_sc as plsc

NBUF = 4
IDXB = 16


def _make_lookup(B: int, L: int, D: int):
    info = plsc.get_sparse_core_info()
    NC, NS = info.num_cores, info.num_subcores
    NW = NC * NS
    DT, DS, BT = D // 8, 8, 128
    NBT = B // BT
    n_blocks = L * NBT
    assert n_blocks % (NW * IDXB) == 0
    blk_per_w = n_blocks // NW
    n_groups = blk_per_w // IDXB
    GSZ = IDXB * BT

    mesh = plsc.VectorSubcoreMesh(core_axis_name="c", subcore_axis_name="s")

    @functools.partial(
        pl.kernel,
        mesh=mesh,
        out_type=jax.ShapeDtypeStruct((L, DT, NBT, DS, BT), jnp.float32),
        scratch_types=[
            pltpu.VMEM((2 * GSZ,), jnp.int32),
            [pltpu.VMEM((BT, D), jnp.float32) for _ in range(NBUF)],
            [pltpu.VMEM((DT, DS, BT + 1), jnp.float32) for _ in range(NBUF)],
            [pltpu.SemaphoreType.DMA for _ in range(NBUF)],
            [pltpu.SemaphoreType.DMA for _ in range(NBUF)],
        ],
        compiler_params=pltpu.CompilerParams(use_tc_tiling_on_sc=False,
                                             needs_layout_passes=False),
    )
    def lookup(idx_hbm, table_hbm, out_hbm, idxv, buf, tbuf, gsem, ssem):
        wid = lax.axis_index("s") * NC + lax.axis_index("c")
        blk0 = wid * blk_per_w

        iota16 = lax.iota(jnp.int32, 16)
        dqs = [iota16 + 16 * q for q in range(D // 16)]
        dtqs = [dq >> 3 for dq in dqs]
        dsqs = [dq & 7 for dq in dqs]

        def stage_idx(g):
            off = (blk0 + g * IDXB) * BT
            pltpu.sync_copy(idx_hbm.at[pl.ds(off, GSZ)],
                            idxv.at[pl.ds((g & 1) * GSZ, GSZ)])

        def gather(blk, slot):
            g = blk >> 4
            t = blk & (IDXB - 1)
            off = (g & 1) * GSZ + t * BT
            src = table_hbm.at[idxv.at[pl.ds(off, BT)]]
            return pltpu.make_async_copy(src, buf[slot], gsem[slot])

        def store(l, bt, dt, slot):
            return pltpu.make_async_copy(tbuf[slot].at[dt, :, pl.ds(0, BT)],
                                         out_hbm.at[l, dt, bt], ssem[slot])

        def transpose(slot):
            @plsc.parallel_loop(0, BT, unroll=8)
            def _(b):
                bvec = jnp.full((16,), b, jnp.int32)
                for q in range(D // 16):
                    v = buf[slot][b, pl.ds(16 * q, 16)]
                    plsc.store_scatter(tbuf[slot], [dtqs[q], dsqs[q], bvec],
                                       v)

        def block_body(s, p):
            blk = s * NBUF + p
            f = blk0 + blk
            l = f >> 7
            bt = f & (NBT - 1)
            gather(blk, p).wait()

            @pl.when(s > 0)
            def _():
                for dt in range(DT):
                    store(l, bt, dt, p).wait()

            transpose(p)
            for dt in range(DT):
                store(l, bt, dt, p).start()

            nblk = blk + NBUF

            @pl.when(nblk < blk_per_w)
            def _():
                gather(nblk, p).start()

        def quadstep(s, _):
            g = s >> 2

            @pl.when((s & 3) == 0)
            def _():
                @pl.when(g < n_groups - 1)
                def _():
                    stage_idx(g + 1)

            for p in range(NBUF):
                block_body(s, p)
            return ()

        stage_idx(jnp.int32(0))
        for p in range(NBUF):
            gather(jnp.int32(p), p).start()

        lax.fori_loop(0, blk_per_w // NBUF, quadstep, (), unroll=False)

        for p in range(NBUF):
            f = blk0 + blk_per_w - NBUF + p
            for dt in range(DT):
                store(f >> 7, f & (NBT - 1), dt, p).wait()

    return lookup


def kernel(p_sequences, table):
    B, L = p_sequences.shape
    V, D = table.shape
    idx_t = p_sequences.T.reshape(B * L)
    lookup = _make_lookup(B, L, D)
    x = lookup(idx_t, table)
    return x.transpose(2, 4, 0, 1, 3).reshape(B, L, D)
